# baseline (device time: 144541 ns/iter reference)
import jax
import jax.numpy as jnp
from jax import lax
from jax.experimental import pallas as pl
from jax.experimental.pallas import tpu as pltpu

N_Z = 4


def kernel(x):
    m, n = x.shape
    qrows = m // 4
    chunk = qrows // N_Z
    sub = chunk // 2

    def body(x_ref, out_ref, comm_ref,
             rs_send, rs_recv, bc_send, bc_recv,
             xq_send, xq_recv, yq_send, yq_recv,
             xf_send, xf_recv, yf_send, yf_recv):
        my_x = lax.axis_index("x")
        my_y = lax.axis_index("y")
        my_z = lax.axis_index("z")
        nxt = (my_z + 1) % N_Z
        h = my_y % 2
        partner = (1 - my_x, my_y, my_z)
        ypair = (my_x, jnp.bitwise_xor(my_y, 1), my_z)

        q_me = 2 * my_x + h
        q_xp = 2 * (1 - my_x) + h
        q_yh = 2 * my_x + (1 - h)

        def rows(q, c, off=0, size=chunk):
            return pl.ds(q * qrows + c * chunk + off, size)

        own = [(my_z + 1) % N_Z, my_z, (my_z - 1) % N_Z, (my_z - 2) % N_Z]

        out_ref[rows(q_me, 0, 0, qrows), :] = x_ref[rows(q_me, 0, 0, qrows), :]

        for s in range(N_Z - 1):
            send_c = (my_z - s) % N_Z
            recv_c = (my_z - s - 1) % N_Z
            rdma = pltpu.make_async_remote_copy(
                src_ref=out_ref.at[rows(q_me, send_c), :],
                dst_ref=comm_ref.at[s],
                send_sem=rs_send.at[s],
                recv_sem=rs_recv.at[s],
                device_id=(my_x, my_y, nxt),
                device_id_type=pl.DeviceIdType.MESH,
            )
            rdma.start()
            rdma.wait()
            out_ref[rows(q_me, recv_c), :] += comm_ref[s]

        xq = [None] * N_Z
        yq = [None] * N_Z
        xf = [None] * N_Z
        yf = [None] * N_Z

        def publish(j):
            c = own[j]
            xq[j] = pltpu.make_async_remote_copy(
                src_ref=out_ref.at[rows(q_me, c), :],
                dst_ref=out_ref.at[rows(q_me, c), :],
                send_sem=xq_send.at[j], recv_sem=xq_recv.at[j],
                device_id=partner, device_id_type=pl.DeviceIdType.MESH,
            )
            xq[j].start()
            yq[j] = pltpu.make_async_remote_copy(
                src_ref=out_ref.at[rows(q_me, c), :],
                dst_ref=out_ref.at[rows(q_me, c), :],
                send_sem=yq_send.at[j], recv_sem=yq_recv.at[j],
                device_id=ypair, device_id_type=pl.DeviceIdType.MESH,
            )
            yq[j].start()

        def forward(j):
            c = own[j]
            yq[j].wait_recv()
            xf[j] = pltpu.make_async_remote_copy(
                src_ref=out_ref.at[rows(q_yh, c, 0, sub), :],
                dst_ref=out_ref.at[rows(q_yh, c, 0, sub), :],
                send_sem=xf_send.at[j], recv_sem=xf_recv.at[j],
                device_id=partner, device_id_type=pl.DeviceIdType.MESH,
            )
            xf[j].start()
            xq[j].wait_recv()
            yf[j] = pltpu.make_async_remote_copy(
                src_ref=out_ref.at[rows(q_xp, c, sub, sub), :],
                dst_ref=out_ref.at[rows(q_xp, c, sub, sub), :],
                send_sem=yf_send.at[j], recv_sem=yf_recv.at[j],
                device_id=ypair, device_id_type=pl.DeviceIdType.MESH,
            )
            yf[j].start()

        publish(0)

        bc = [None] * N_Z
        for d in (1, 2, 3):
            tz = (my_z + d) % N_Z
            bc[d] = pltpu.make_async_remote_copy(
                src_ref=out_ref.at[rows(q_me, own[0]), :],
                dst_ref=out_ref.at[rows(q_me, own[0]), :],
                send_sem=bc_send.at[d - 1], recv_sem=bc_recv.at[d - 1],
                device_id=(my_x, my_y, tz),
                device_id_type=pl.DeviceIdType.MESH,
            )
            bc[d].start()
        for j in (1, 2, 3):
            bc[j].wait_recv()
            publish(j)

        for j in range(N_Z):
            forward(j)

        for d in (1, 2, 3):
            bc[d].wait_send()
        for j in range(N_Z):
            xq[j].wait_send()
            yq[j].wait_send()
            xf[j].wait()
            yf[j].wait()

    return pl.pallas_call(
        body,
        out_shape=jax.ShapeDtypeStruct((m, n), x.dtype),
        in_specs=[pl.BlockSpec(memory_space=pltpu.VMEM)],
        out_specs=pl.BlockSpec(memory_space=pltpu.VMEM),
        scratch_shapes=[
            pltpu.VMEM((N_Z - 1, chunk, n), x.dtype),
            pltpu.SemaphoreType.DMA((N_Z - 1,)),
            pltpu.SemaphoreType.DMA((N_Z - 1,)),
            pltpu.SemaphoreType.DMA((N_Z - 1,)),
            pltpu.SemaphoreType.DMA((N_Z - 1,)),
            pltpu.SemaphoreType.DMA((N_Z,)),
            pltpu.SemaphoreType.DMA((N_Z,)),
            pltpu.SemaphoreType.DMA((N_Z,)),
            pltpu.SemaphoreType.DMA((N_Z,)),
            pltpu.SemaphoreType.DMA((N_Z,)),
            pltpu.SemaphoreType.DMA((N_Z,)),
            pltpu.SemaphoreType.DMA((N_Z,)),
            pltpu.SemaphoreType.DMA((N_Z,)),
        ],
    )(x)


# device time: 130000 ns/iter; 1.1119x vs baseline; 1.1119x over previous
import jax
import jax.numpy as jnp
from jax import lax
from jax.experimental import pallas as pl
from jax.experimental.pallas import tpu as pltpu

N_Z = 4
N_STR = 2


def kernel(x):
    m, n = x.shape
    qrows = m // 4
    chunk = qrows // N_Z
    sub = chunk // N_STR

    def body(x_ref, out_ref, comm_ref,
             rs_send, rs_recv, ag_send, ag_recv,
             xq_send, xq_recv, yq_send, yq_recv,
             xf_send, xf_recv, yf_send, yf_recv):
        my_x = lax.axis_index("x")
        my_y = lax.axis_index("y")
        my_z = lax.axis_index("z")
        nxt = (my_z + 1) % N_Z
        h = my_y % 2
        partner = (1 - my_x, my_y, my_z)
        ypair = (my_x, jnp.bitwise_xor(my_y, 1), my_z)

        q_me = 2 * my_x + h
        q_xp = 2 * (1 - my_x) + h
        q_yh = 2 * my_x + (1 - h)

        def rows(q, c, g, size=sub):
            return pl.ds(q * qrows + c * chunk + g * sub, size)

        own = [(my_z + 1) % N_Z, my_z, (my_z - 1) % N_Z, (my_z - 2) % N_Z]

        out_ref[pl.ds(q_me * qrows, qrows), :] = x_ref[pl.ds(q_me * qrows, qrows), :]

        rs = [[None] * N_STR for _ in range(N_Z - 1)]

        def rs_start(s, g):
            send_c = (my_z - s) % N_Z
            rs[s][g] = pltpu.make_async_remote_copy(
                src_ref=out_ref.at[rows(q_me, send_c, g), :],
                dst_ref=comm_ref.at[s * N_STR + g],
                send_sem=rs_send.at[s * N_STR + g],
                recv_sem=rs_recv.at[s * N_STR + g],
                device_id=(my_x, my_y, nxt),
                device_id_type=pl.DeviceIdType.MESH,
            )
            rs[s][g].start()

        rs_start(0, 0)
        rs_start(0, 1)
        for s in range(N_Z - 1):
            recv_c = (my_z - s - 1) % N_Z
            for g in range(N_STR):
                rs[s][g].wait()
                out_ref[rows(q_me, recv_c, g), :] += comm_ref[s * N_STR + g]
                if s < N_Z - 2:
                    rs_start(s + 1, g)

        xq = [[None] * N_STR for _ in range(N_Z)]
        yq = [[None] * N_STR for _ in range(N_Z)]
        xf = [None] * N_Z
        yf = [None] * N_Z

        def publish(j, g):
            c = own[j]
            xq[j][g] = pltpu.make_async_remote_copy(
                src_ref=out_ref.at[rows(q_me, c, g), :],
                dst_ref=out_ref.at[rows(q_me, c, g), :],
                send_sem=xq_send.at[j * N_STR + g],
                recv_sem=xq_recv.at[j * N_STR + g],
                device_id=partner, device_id_type=pl.DeviceIdType.MESH,
            )
            xq[j][g].start()
            yq[j][g] = pltpu.make_async_remote_copy(
                src_ref=out_ref.at[rows(q_me, c, g), :],
                dst_ref=out_ref.at[rows(q_me, c, g), :],
                send_sem=yq_send.at[j * N_STR + g],
                recv_sem=yq_recv.at[j * N_STR + g],
                device_id=ypair, device_id_type=pl.DeviceIdType.MESH,
            )
            yq[j][g].start()

        ag = [[None] * N_STR for _ in range(N_Z - 1)]

        def ag_start(s, g):
            c = own[s]
            ag[s][g] = pltpu.make_async_remote_copy(
                src_ref=out_ref.at[rows(q_me, c, g), :],
                dst_ref=out_ref.at[rows(q_me, c, g), :],
                send_sem=ag_send.at[s * N_STR + g],
                recv_sem=ag_recv.at[s * N_STR + g],
                device_id=(my_x, my_y, nxt),
                device_id_type=pl.DeviceIdType.MESH,
            )
            ag[s][g].start()

        publish(0, 0)
        publish(0, 1)
        ag_start(0, 0)
        ag_start(0, 1)
        for s in range(N_Z - 1):
            for g in range(N_STR):
                ag[s][g].wait()
                publish(s + 1, g)
                if s < N_Z - 2:
                    ag_start(s + 1, g)

        for j in range(N_Z):
            c = own[j]
            yq[j][0].wait_recv()
            xf[j] = pltpu.make_async_remote_copy(
                src_ref=out_ref.at[rows(q_yh, c, 0), :],
                dst_ref=out_ref.at[rows(q_yh, c, 0), :],
                send_sem=xf_send.at[j], recv_sem=xf_recv.at[j],
                device_id=partner, device_id_type=pl.DeviceIdType.MESH,
            )
            xf[j].start()
            xq[j][1].wait_recv()
            yf[j] = pltpu.make_async_remote_copy(
                src_ref=out_ref.at[rows(q_xp, c, 1), :],
                dst_ref=out_ref.at[rows(q_xp, c, 1), :],
                send_sem=yf_send.at[j], recv_sem=yf_recv.at[j],
                device_id=ypair, device_id_type=pl.DeviceIdType.MESH,
            )
            yf[j].start()

        for j in range(N_Z):
            xq[j][0].wait()
            xq[j][1].wait_send()
            yq[j][0].wait_send()
            yq[j][1].wait()
            xf[j].wait()
            yf[j].wait()

    return pl.pallas_call(
        body,
        out_shape=jax.ShapeDtypeStruct((m, n), x.dtype),
        in_specs=[pl.BlockSpec(memory_space=pltpu.VMEM)],
        out_specs=pl.BlockSpec(memory_space=pltpu.VMEM),
        scratch_shapes=[
            pltpu.VMEM(((N_Z - 1) * N_STR, sub, n), x.dtype),
            pltpu.SemaphoreType.DMA(((N_Z - 1) * N_STR,)),
            pltpu.SemaphoreType.DMA(((N_Z - 1) * N_STR,)),
            pltpu.SemaphoreType.DMA(((N_Z - 1) * N_STR,)),
            pltpu.SemaphoreType.DMA(((N_Z - 1) * N_STR,)),
            pltpu.SemaphoreType.DMA((N_Z * N_STR,)),
            pltpu.SemaphoreType.DMA((N_Z * N_STR,)),
            pltpu.SemaphoreType.DMA((N_Z * N_STR,)),
            pltpu.SemaphoreType.DMA((N_Z * N_STR,)),
            pltpu.SemaphoreType.DMA((N_Z,)),
            pltpu.SemaphoreType.DMA((N_Z,)),
            pltpu.SemaphoreType.DMA((N_Z,)),
            pltpu.SemaphoreType.DMA((N_Z,)),
        ],
    )(x)


# device time: 129976 ns/iter; 1.1121x vs baseline; 1.0002x over previous
import jax
import jax.numpy as jnp
from jax import lax
from jax.experimental import pallas as pl
from jax.experimental.pallas import tpu as pltpu

N_Z = 4
N_STR = 2


def kernel(x):
    m, n = x.shape
    qrows = m // 4
    chunk = qrows // N_Z
    sub = chunk // N_STR

    def body(x_ref, out_ref, comm_ref,
             rs_send, rs_recv, ag_send, ag_recv,
             xq_send, xq_recv, yq_send, yq_recv,
             xf_send, xf_recv, yf_send, yf_recv):
        my_x = lax.axis_index("x")
        my_y = lax.axis_index("y")
        my_z = lax.axis_index("z")
        nxt = (my_z + 1) % N_Z
        h = my_y % 2
        partner = (1 - my_x, my_y, my_z)
        ypair = (my_x, jnp.bitwise_xor(my_y, 1), my_z)

        q_me = 2 * my_x + h
        q_xp = 2 * (1 - my_x) + h
        q_yh = 2 * my_x + (1 - h)

        def rows(q, c, g, size=sub):
            return pl.ds(q * qrows + c * chunk + g * sub, size)

        own = [(my_z + 1) % N_Z, my_z, (my_z - 1) % N_Z, (my_z - 2) % N_Z]

        out_ref[pl.ds(q_me * qrows, qrows), :] = x_ref[pl.ds(q_me * qrows, qrows), :]

        rs = [[None] * N_STR for _ in range(N_Z - 1)]

        def rs_start(s, g):
            send_c = (my_z - s) % N_Z
            rs[s][g] = pltpu.make_async_remote_copy(
                src_ref=out_ref.at[rows(q_me, send_c, g), :],
                dst_ref=comm_ref.at[s * N_STR + g],
                send_sem=rs_send.at[s * N_STR + g],
                recv_sem=rs_recv.at[s * N_STR + g],
                device_id=(my_x, my_y, nxt),
                device_id_type=pl.DeviceIdType.MESH,
            )
            rs[s][g].start()

        rs_start(0, 0)
        rs_start(0, 1)
        for s in range(N_Z - 1):
            recv_c = (my_z - s - 1) % N_Z
            for g in range(N_STR):
                rs[s][g].wait()
                out_ref[rows(q_me, recv_c, g), :] += comm_ref[s * N_STR + g]
                if s < N_Z - 2:
                    rs_start(s + 1, g)

        xq = [[None] * N_STR for _ in range(N_Z)]
        yq = [[None] * N_STR for _ in range(N_Z)]
        xf = [None] * N_Z
        yf = [None] * N_Z

        def publish(j, g):
            c = own[j]
            xq[j][g] = pltpu.make_async_remote_copy(
                src_ref=out_ref.at[rows(q_me, c, g), :],
                dst_ref=out_ref.at[rows(q_me, c, g), :],
                send_sem=xq_send.at[j * N_STR + g],
                recv_sem=xq_recv.at[j * N_STR + g],
                device_id=partner, device_id_type=pl.DeviceIdType.MESH,
            )
            xq[j][g].start()
            yq[j][g] = pltpu.make_async_remote_copy(
                src_ref=out_ref.at[rows(q_me, c, g), :],
                dst_ref=out_ref.at[rows(q_me, c, g), :],
                send_sem=yq_send.at[j * N_STR + g],
                recv_sem=yq_recv.at[j * N_STR + g],
                device_id=ypair, device_id_type=pl.DeviceIdType.MESH,
            )
            yq[j][g].start()

        ag = [[None] * N_STR for _ in range(N_Z - 1)]

        def ag_start(s, g):
            c = own[s]
            ag[s][g] = pltpu.make_async_remote_copy(
                src_ref=out_ref.at[rows(q_me, c, g), :],
                dst_ref=out_ref.at[rows(q_me, c, g), :],
                send_sem=ag_send.at[s * N_STR + g],
                recv_sem=ag_recv.at[s * N_STR + g],
                device_id=(my_x, my_y, nxt),
                device_id_type=pl.DeviceIdType.MESH,
            )
            ag[s][g].start()

        def forward(j):
            c = own[j]
            yq[j][0].wait_recv()
            xf[j] = pltpu.make_async_remote_copy(
                src_ref=out_ref.at[rows(q_yh, c, 0), :],
                dst_ref=out_ref.at[rows(q_yh, c, 0), :],
                send_sem=xf_send.at[j], recv_sem=xf_recv.at[j],
                device_id=partner, device_id_type=pl.DeviceIdType.MESH,
            )
            xf[j].start()
            xq[j][1].wait_recv()
            yf[j] = pltpu.make_async_remote_copy(
                src_ref=out_ref.at[rows(q_xp, c, 1), :],
                dst_ref=out_ref.at[rows(q_xp, c, 1), :],
                send_sem=yf_send.at[j], recv_sem=yf_recv.at[j],
                device_id=ypair, device_id_type=pl.DeviceIdType.MESH,
            )
            yf[j].start()

        publish(0, 0)
        publish(0, 1)
        ag_start(0, 0)
        ag_start(0, 1)
        for s in range(N_Z - 1):
            for g in range(N_STR):
                ag[s][g].wait()
                publish(s + 1, g)
                if s < N_Z - 2:
                    ag_start(s + 1, g)
            forward(s)
        forward(N_Z - 1)

        for j in range(N_Z):
            xq[j][0].wait()
            xq[j][1].wait_send()
            yq[j][0].wait_send()
            yq[j][1].wait()
            xf[j].wait()
            yf[j].wait()

    return pl.pallas_call(
        body,
        out_shape=jax.ShapeDtypeStruct((m, n), x.dtype),
        in_specs=[pl.BlockSpec(memory_space=pltpu.VMEM)],
        out_specs=pl.BlockSpec(memory_space=pltpu.VMEM),
        scratch_shapes=[
            pltpu.VMEM(((N_Z - 1) * N_STR, sub, n), x.dtype),
            pltpu.SemaphoreType.DMA(((N_Z - 1) * N_STR,)),
            pltpu.SemaphoreType.DMA(((N_Z - 1) * N_STR,)),
            pltpu.SemaphoreType.DMA(((N_Z - 1) * N_STR,)),
            pltpu.SemaphoreType.DMA(((N_Z - 1) * N_STR,)),
            pltpu.SemaphoreType.DMA((N_Z * N_STR,)),
            pltpu.SemaphoreType.DMA((N_Z * N_STR,)),
            pltpu.SemaphoreType.DMA((N_Z * N_STR,)),
            pltpu.SemaphoreType.DMA((N_Z * N_STR,)),
            pltpu.SemaphoreType.DMA((N_Z,)),
            pltpu.SemaphoreType.DMA((N_Z,)),
            pltpu.SemaphoreType.DMA((N_Z,)),
            pltpu.SemaphoreType.DMA((N_Z,)),
        ],
    )(x)
